# baseline (speedup 1.0000x reference)
import functools

import jax
import jax.numpy as jnp
from jax import lax
from jax.experimental import pallas as pl
from jax.experimental.pallas import tpu as pltpu
from jax.experimental.pallas import tpu_sc as plsc

_B = 16384
_D = 256
_R = 2048
_NB = _B // _R

_NW = 32
_C = _B // _NW
_CC = 128
_NCH = _C // _CC


def _gelu_exact(x):
    return 0.5 * x * (1.0 + lax.erf(x * (2.0 ** -0.5)))


_sc_mesh = plsc.VectorSubcoreMesh(core_axis_name="c", subcore_axis_name="s")


@functools.partial(
    pl.kernel,
    mesh=_sc_mesh,
    out_type=jax.ShapeDtypeStruct((_B, 4, _D), jnp.float32),
    scratch_types=[
        pltpu.VMEM((_CC,), jnp.int32),
        pltpu.VMEM((_CC,), jnp.int32),
        pltpu.VMEM((_CC, _D), jnp.float32),
        pltpu.VMEM((_CC, _D), jnp.float32),
        pltpu.SemaphoreType.DMA,
        pltpu.SemaphoreType.DMA,
    ],
)
def _sc_writer(gidx_hbm, aidx_hbm, gtab_hbm, atab_hbm, out_hbm,
               gidx_v, aidx_v, grow_v, arow_v, gsem, asem):
    wid = lax.axis_index("s") * 2 + lax.axis_index("c")
    base = wid * _C
    for t in range(_NCH):
        b0 = base + t * _CC
        pltpu.sync_copy(gidx_hbm.at[pl.ds(b0, _CC)], gidx_v)
        pltpu.sync_copy(aidx_hbm.at[pl.ds(b0, _CC)], aidx_v)
        pltpu.async_copy(gtab_hbm.at[gidx_v], grow_v, gsem).wait()
        pltpu.async_copy(atab_hbm.at[aidx_v], arow_v, asem).wait()
        pltpu.sync_copy(grow_v, out_hbm.at[pl.ds(b0, _CC), 1, :])
        pltpu.sync_copy(arow_v, out_hbm.at[pl.ds(b0, _CC), 2, :])


def _tc_body(partial_ref, xt_ref, cls_ref, bmb_ref,
             w1_ref, b1_ref, w2_ref, b2_ref, w3_ref, b3_ref, out_ref,
             buf_ref, sem_ref):
    i = pl.program_id(0)
    s = lax.rem(i, 2)
    r = _R

    def copies(step, slot, plane):
        return pltpu.make_async_copy(
            buf_ref.at[lax.rem(step, 2), plane],
            out_ref.at[pl.ds(step * _R, _R), slot, :],
            sem_ref.at[lax.rem(step, 2), plane])

    @pl.when(i >= 2)
    def _():
        copies(i - 2, 0, 0).wait()
        copies(i - 2, 3, 1).wait()

    buf_ref[s, 0] = jnp.broadcast_to(cls_ref[...], (r, _D))
    copies(i, 0, 0).start()

    h = lax.dot_general(xt_ref[...], w1_ref[...], (((0,), (0,)), ((), ())),
                        preferred_element_type=jnp.float32) + b1_ref[...]
    h = _gelu_exact(h)
    h = jnp.dot(h, w2_ref[...], preferred_element_type=jnp.float32) + b2_ref[...]
    h = _gelu_exact(h)
    h = jnp.dot(h, w3_ref[...], preferred_element_type=jnp.float32) + b3_ref[...]
    buf_ref[s, 1] = h + bmb_ref[...]
    copies(i, 3, 1).start()

    @pl.when(i == _NB - 1)
    def _():
        copies(i - 1, 0, 0).wait()
        copies(i - 1, 3, 1).wait()
        copies(i, 0, 0).wait()
        copies(i, 3, 1).wait()


def kernel(user_gender, user_age_bin, user_born_mort, cls_param, gender_table,
           age_table, born_mort_bias, W1, b1, W2, b2, W3, b3):
    n = user_born_mort.shape[0]
    xt = user_born_mort.T

    partial_out = _sc_writer(user_gender.astype(jnp.int32),
                             user_age_bin.astype(jnp.int32),
                             gender_table, age_table)

    full = lambda shape: pl.BlockSpec(shape, lambda i: (0,) * len(shape))
    out3d = pl.pallas_call(
        _tc_body,
        grid=(_NB,),
        in_specs=[
            pl.BlockSpec(memory_space=pl.ANY),
            pl.BlockSpec((13, _R), lambda i: (0, i)),
            full((1, _D)),
            full((1, _D)),
            full((13, 64)),
            full((1, 64)),
            full((64, 128)),
            full((1, 128)),
            full((128, _D)),
            full((1, _D)),
        ],
        out_specs=pl.BlockSpec(memory_space=pl.ANY),
        out_shape=jax.ShapeDtypeStruct((n, 4, _D), jnp.float32),
        input_output_aliases={0: 0},
        scratch_shapes=[
            pltpu.VMEM((2, 2, _R, _D), jnp.float32),
            pltpu.SemaphoreType.DMA((2, 2)),
        ],
        compiler_params=pltpu.CompilerParams(
            dimension_semantics=("arbitrary",)),
    )(partial_out, xt, cls_param, born_mort_bias,
      W1, b1.reshape(1, 64), W2, b2.reshape(1, 128), W3, b3.reshape(1, _D))

    mask = jnp.ones((n, 4), dtype=jnp.int32)
    return (out3d, mask)


# speedup vs baseline: 18.0064x; 18.0064x over previous
import jax
import jax.numpy as jnp
from jax import lax
from jax.experimental import pallas as pl
from jax.experimental.pallas import tpu as pltpu

_B = 16384
_D = 256
_R = 2048
_NB = _B // _R


def _gelu_exact(x):
    return 0.5 * x * (1.0 + lax.erf(x * (2.0 ** -0.5)))


def _body(gidx_ref, aidx_ref, xt_ref, cls_ref, gtab_ref, atab_ref, bmb_ref,
          w1_ref, b1_ref, w2_ref, b2_ref, w3_ref, b3_ref, out_ref,
          buf_ref, sem_ref):
    i = pl.program_id(0)
    s = lax.rem(i, 3)
    r = _R

    def copies(step, slot):
        return pltpu.make_async_copy(
            buf_ref.at[lax.rem(step, 3), slot],
            out_ref.at[pl.ds(step * _R, _R), slot, :],
            sem_ref.at[lax.rem(step, 3), slot])

    @pl.when(i >= 3)
    def _():
        for j in range(4):
            copies(i - 3, j).wait()

    buf_ref[s, 0] = jnp.broadcast_to(cls_ref[...], (r, _D))
    copies(i, 0).start()

    g = gidx_ref[0, 0, :]
    goh = (g[:, None] == lax.broadcasted_iota(jnp.int32, (r, 2), 1)
           ).astype(jnp.float32)
    buf_ref[s, 1] = jnp.dot(goh, gtab_ref[...],
                            preferred_element_type=jnp.float32)
    copies(i, 1).start()

    a = aidx_ref[0, 0, :]
    aoh = (a[:, None] == lax.broadcasted_iota(jnp.int32, (r, 7), 1)
           ).astype(jnp.float32)
    buf_ref[s, 2] = jnp.dot(aoh, atab_ref[...],
                            preferred_element_type=jnp.float32)
    copies(i, 2).start()

    h = lax.dot_general(xt_ref[...], w1_ref[...], (((0,), (0,)), ((), ())),
                        preferred_element_type=jnp.float32) + b1_ref[...]
    h = _gelu_exact(h)
    h = jnp.dot(h, w2_ref[...], preferred_element_type=jnp.float32) + b2_ref[...]
    h = _gelu_exact(h)
    h = jnp.dot(h, w3_ref[...], preferred_element_type=jnp.float32) + b3_ref[...]
    buf_ref[s, 3] = h + bmb_ref[...]
    copies(i, 3).start()

    @pl.when(i == _NB - 1)
    def _():
        for j in range(4):
            copies(i - 2, j).wait()
        for j in range(4):
            copies(i - 1, j).wait()
        for j in range(4):
            copies(i, j).wait()


def kernel(user_gender, user_age_bin, user_born_mort, cls_param, gender_table,
           age_table, born_mort_bias, W1, b1, W2, b2, W3, b3):
    n = user_born_mort.shape[0]
    gidx = user_gender.astype(jnp.int32).reshape(_NB, 1, _R)
    aidx = user_age_bin.astype(jnp.int32).reshape(_NB, 1, _R)
    xt = user_born_mort.T

    full = lambda shape: pl.BlockSpec(shape, lambda i: (0,) * len(shape))
    out3d = pl.pallas_call(
        _body,
        grid=(_NB,),
        in_specs=[
            pl.BlockSpec((1, 1, _R), lambda i: (i, 0, 0)),
            pl.BlockSpec((1, 1, _R), lambda i: (i, 0, 0)),
            pl.BlockSpec((13, _R), lambda i: (0, i)),
            full((1, _D)),
            full((2, _D)),
            full((7, _D)),
            full((1, _D)),
            full((13, 64)),
            full((1, 64)),
            full((64, 128)),
            full((1, 128)),
            full((128, _D)),
            full((1, _D)),
        ],
        out_specs=pl.BlockSpec(memory_space=pl.ANY),
        out_shape=jax.ShapeDtypeStruct((n, 4, _D), jnp.float32),
        scratch_shapes=[
            pltpu.VMEM((3, 4, _R, _D), jnp.float32),
            pltpu.SemaphoreType.DMA((3, 4)),
        ],
        compiler_params=pltpu.CompilerParams(
            dimension_semantics=("arbitrary",)),
    )(gidx, aidx, xt, cls_param, gender_table, age_table,
      born_mort_bias, W1, b1.reshape(1, 64), W2, b2.reshape(1, 128),
      W3, b3.reshape(1, _D))

    mask = jnp.ones((n, 4), dtype=jnp.int32)
    return (out3d, mask)


# speedup vs baseline: 18.0713x; 1.0036x over previous
import jax
import jax.numpy as jnp
from jax import lax
from jax.experimental import pallas as pl
from jax.experimental.pallas import tpu as pltpu

_B = 16384
_D = 256
_R = 2048
_NB = _B // _R


def _gelu_exact(x):
    return 0.5 * x * (1.0 + lax.erf(x * (2.0 ** -0.5)))


def _body(gidx_ref, aidx_ref, xt_ref, cls_ref, gtab_ref, atab_ref, bmb_ref,
          w1_ref, b1_ref, w2_ref, b2_ref, w3_ref, b3_ref, out_ref,
          buf_ref, sem_ref):
    i = pl.program_id(0)
    s = lax.rem(i, 2)
    r = _R

    def copies(step, slot):
        return pltpu.make_async_copy(
            buf_ref.at[lax.rem(step, 2), slot],
            out_ref.at[pl.ds(step * _R, _R), slot, :],
            sem_ref.at[lax.rem(step, 2), slot])

    @pl.when(i >= 2)
    def _():
        for j in range(4):
            copies(i - 2, j).wait()

    buf_ref[s, 0] = jnp.broadcast_to(cls_ref[...], (r, _D))
    copies(i, 0).start()

    g = gidx_ref[0, 0, :]
    goh = (g[:, None] == lax.broadcasted_iota(jnp.int32, (r, 2), 1)
           ).astype(jnp.float32)
    buf_ref[s, 1] = jnp.dot(goh, gtab_ref[...],
                            preferred_element_type=jnp.float32)
    copies(i, 1).start()

    a = aidx_ref[0, 0, :]
    aoh = (a[:, None] == lax.broadcasted_iota(jnp.int32, (r, 7), 1)
           ).astype(jnp.float32)
    buf_ref[s, 2] = jnp.dot(aoh, atab_ref[...],
                            preferred_element_type=jnp.float32)
    copies(i, 2).start()

    h = lax.dot_general(xt_ref[...], w1_ref[...], (((0,), (0,)), ((), ())),
                        preferred_element_type=jnp.float32) + b1_ref[...]
    h = _gelu_exact(h)
    h = jnp.dot(h, w2_ref[...], preferred_element_type=jnp.float32) + b2_ref[...]
    h = _gelu_exact(h)
    h = jnp.dot(h, w3_ref[...], preferred_element_type=jnp.float32) + b3_ref[...]
    buf_ref[s, 3] = h + bmb_ref[...]
    copies(i, 3).start()

    @pl.when(i == _NB - 1)
    def _():
        for j in range(4):
            copies(i - 1, j).wait()
        for j in range(4):
            copies(i, j).wait()


def kernel(user_gender, user_age_bin, user_born_mort, cls_param, gender_table,
           age_table, born_mort_bias, W1, b1, W2, b2, W3, b3):
    n = user_born_mort.shape[0]
    gidx = user_gender.astype(jnp.int32).reshape(_NB, 1, _R)
    aidx = user_age_bin.astype(jnp.int32).reshape(_NB, 1, _R)
    xt = user_born_mort.T

    full = lambda shape: pl.BlockSpec(shape, lambda i: (0,) * len(shape))
    out3d = pl.pallas_call(
        _body,
        grid=(_NB,),
        in_specs=[
            pl.BlockSpec((1, 1, _R), lambda i: (i, 0, 0)),
            pl.BlockSpec((1, 1, _R), lambda i: (i, 0, 0)),
            pl.BlockSpec((13, _R), lambda i: (0, i)),
            full((1, _D)),
            full((2, _D)),
            full((7, _D)),
            full((1, _D)),
            full((13, 64)),
            full((1, 64)),
            full((64, 128)),
            full((1, 128)),
            full((128, _D)),
            full((1, _D)),
        ],
        out_specs=pl.BlockSpec(memory_space=pl.ANY),
        out_shape=jax.ShapeDtypeStruct((n, 4, _D), jnp.float32),
        scratch_shapes=[
            pltpu.VMEM((2, 4, _R, _D), jnp.float32),
            pltpu.SemaphoreType.DMA((2, 4)),
        ],
        compiler_params=pltpu.CompilerParams(
            dimension_semantics=("arbitrary",)),
    )(gidx, aidx, xt, cls_param, gender_table, age_table,
      born_mort_bias, W1, b1.reshape(1, 64), W2, b2.reshape(1, 128),
      W3, b3.reshape(1, _D))

    mask = jnp.ones((n, 4), dtype=jnp.int32)
    return (out3d, mask)
